# baseline (device time: 49070 ns/iter reference)
import jax
import jax.numpy as jnp
from jax import lax
from jax.experimental import pallas as pl
from jax.experimental.pallas import tpu as pltpu

N_DEV = 16
LOG2_N = 4
B, SQ, SKV, HQ_SH, DH = 2, 128, 128, 4, 64
D_MODEL = 512


def kernel(x, Wq, K_ext, V_ext, Wo):
    my_i = lax.axis_index("i")
    K_sh = lax.dynamic_slice_in_dim(K_ext, my_i * HQ_SH, HQ_SH, axis=2)
    V_sh = lax.dynamic_slice_in_dim(V_ext, my_i * HQ_SH, HQ_SH, axis=2)
    K_sh = jnp.transpose(K_sh, (0, 2, 1, 3))
    V_sh = jnp.transpose(V_sh, (0, 2, 1, 3))

    def body(x_ref, wq_ref, k_ref, v_ref, wo_ref, out_ref,
             ctx_ref, recv_ref, send_sems, recv_sems):
        me = lax.axis_index("i")

        wq = wq_ref[...].astype(jnp.bfloat16)
        wo = wo_ref[...].astype(jnp.bfloat16)
        ri = lax.broadcasted_iota(jnp.int32, (SQ, SKV), 0)
        ci = lax.broadcasted_iota(jnp.int32, (SQ, SKV), 1)
        qb = ri // 64
        kb = ci // 64
        mask = (qb == kb) | ((kb % 4) == (qb % 4))
        for b in range(B):
            xb = x_ref[b].astype(jnp.bfloat16)
            q_all = jnp.dot(xb, wq, preferred_element_type=jnp.float32)
            q_all = q_all.astype(jnp.bfloat16)
            for h in range(HQ_SH):
                q = q_all[:, h * DH:(h + 1) * DH]
                k = k_ref[b, h].astype(jnp.bfloat16)
                s = lax.dot_general(q, k, (((1,), (1,)), ((), ())),
                                    preferred_element_type=jnp.float32)
                s = s * 0.125
                s = jnp.where(mask, s, -1e9)
                m = jnp.max(s, axis=-1, keepdims=True)
                w = jnp.exp(s - m)
                w = w / jnp.sum(w, axis=-1, keepdims=True)
                v = v_ref[b, h].astype(jnp.bfloat16)
                ctx = jnp.dot(w.astype(jnp.bfloat16), v,
                              preferred_element_type=jnp.float32)
                ctx_ref[b, :, h * DH:(h + 1) * DH] = ctx.astype(jnp.bfloat16)
            out_ref[b] = jnp.dot(ctx_ref[b], wo,
                                 preferred_element_type=jnp.float32)

        barrier = pltpu.get_barrier_semaphore()
        for r in range(LOG2_N):
            partner = me ^ (1 << r)
            pl.semaphore_signal(barrier, inc=1, device_id=(partner,),
                                device_id_type=pl.DeviceIdType.MESH)
        pl.semaphore_wait(barrier, LOG2_N)

        for r in range(LOG2_N):
            partner = me ^ (1 << r)
            rdma = pltpu.make_async_remote_copy(
                src_ref=out_ref,
                dst_ref=recv_ref.at[r],
                send_sem=send_sems.at[r],
                recv_sem=recv_sems.at[r],
                device_id=(partner,),
                device_id_type=pl.DeviceIdType.MESH,
            )
            rdma.start()
            rdma.wait()
            out_ref[...] = out_ref[...] + recv_ref[r]

    return pl.pallas_call(
        body,
        out_shape=jax.ShapeDtypeStruct((B, SQ, D_MODEL), jnp.float32),
        in_specs=[pl.BlockSpec(memory_space=pltpu.VMEM)] * 5,
        out_specs=pl.BlockSpec(memory_space=pltpu.VMEM),
        scratch_shapes=[
            pltpu.VMEM((B, SQ, HQ_SH * DH), jnp.bfloat16),
            pltpu.VMEM((LOG2_N, B, SQ, D_MODEL), jnp.float32),
            pltpu.SemaphoreType.DMA((LOG2_N,)),
            pltpu.SemaphoreType.DMA((LOG2_N,)),
        ],
        compiler_params=pltpu.CompilerParams(collective_id=0),
    )(x, Wq, K_sh, V_sh, Wo)


# device time: 35153 ns/iter; 1.3959x vs baseline; 1.3959x over previous
import jax
import jax.numpy as jnp
from jax import lax
from jax.experimental import pallas as pl
from jax.experimental.pallas import tpu as pltpu

N_DEV = 16
LOG2_N = 4
B, SQ, SKV, HQ_SH, DH = 2, 128, 128, 4, 64
D_MODEL = 512


def kernel(x, Wq, K_ext, V_ext, Wo):
    my_i = lax.axis_index("i")
    K_sh = lax.dynamic_slice_in_dim(K_ext, my_i * HQ_SH, HQ_SH, axis=2)
    V_sh = lax.dynamic_slice_in_dim(V_ext, my_i * HQ_SH, HQ_SH, axis=2)
    K_sh = jnp.transpose(K_sh, (0, 2, 1, 3))
    V_sh = jnp.transpose(V_sh, (0, 2, 1, 3))

    def body(x_ref, wq_ref, k_ref, v_ref, wo_ref, out_ref,
             ctx_ref, send_ref, recv_ref, send_sems, recv_sems):
        me = lax.axis_index("i")

        wq = wq_ref[...].astype(jnp.bfloat16)
        wo = wo_ref[...].astype(jnp.bfloat16)
        ri = lax.broadcasted_iota(jnp.int32, (SQ, SKV), 0)
        ci = lax.broadcasted_iota(jnp.int32, (SQ, SKV), 1)
        qb = ri // 64
        kb = ci // 64
        mask = (qb == kb) | ((kb % 4) == (qb % 4))
        for b in range(B):
            xb = x_ref[b].astype(jnp.bfloat16)
            q_all = jnp.dot(xb, wq, preferred_element_type=jnp.float32)
            q_all = q_all.astype(jnp.bfloat16)
            for h in range(HQ_SH):
                q = q_all[:, h * DH:(h + 1) * DH]
                k = k_ref[b, h].astype(jnp.bfloat16)
                s = lax.dot_general(q, k, (((1,), (1,)), ((), ())),
                                    preferred_element_type=jnp.float32)
                s = s * 0.125
                s = jnp.where(mask, s, -1e9)
                m = jnp.max(s, axis=-1, keepdims=True)
                w = jnp.exp(s - m)
                w = w / jnp.sum(w, axis=-1, keepdims=True)
                v = v_ref[b, h].astype(jnp.bfloat16)
                ctx = jnp.dot(w.astype(jnp.bfloat16), v,
                              preferred_element_type=jnp.float32)
                ctx_ref[b, :, h * DH:(h + 1) * DH] = ctx.astype(jnp.bfloat16)
            out_ref[b] = jnp.dot(ctx_ref[b], wo,
                                 preferred_element_type=jnp.float32)

        barrier = pltpu.get_barrier_semaphore()
        for r in range(LOG2_N):
            partner = me ^ (1 << r)
            pl.semaphore_signal(barrier, inc=1, device_id=(partner,),
                                device_id_type=pl.DeviceIdType.MESH)
        pl.semaphore_wait(barrier, LOG2_N)

        for r in range(LOG2_N):
            partner = me ^ (1 << r)
            send_ref[...] = out_ref[...].astype(jnp.bfloat16)
            rdma = pltpu.make_async_remote_copy(
                src_ref=send_ref,
                dst_ref=recv_ref.at[r],
                send_sem=send_sems.at[r],
                recv_sem=recv_sems.at[r],
                device_id=(partner,),
                device_id_type=pl.DeviceIdType.MESH,
            )
            rdma.start()
            rdma.wait()
            out_ref[...] = out_ref[...] + recv_ref[r].astype(jnp.float32)

    return pl.pallas_call(
        body,
        out_shape=jax.ShapeDtypeStruct((B, SQ, D_MODEL), jnp.float32),
        in_specs=[pl.BlockSpec(memory_space=pltpu.VMEM)] * 5,
        out_specs=pl.BlockSpec(memory_space=pltpu.VMEM),
        scratch_shapes=[
            pltpu.VMEM((B, SQ, HQ_SH * DH), jnp.bfloat16),
            pltpu.VMEM((B, SQ, D_MODEL), jnp.bfloat16),
            pltpu.VMEM((LOG2_N, B, SQ, D_MODEL), jnp.bfloat16),
            pltpu.SemaphoreType.DMA((LOG2_N,)),
            pltpu.SemaphoreType.DMA((LOG2_N,)),
        ],
        compiler_params=pltpu.CompilerParams(collective_id=0),
    )(x, Wq, K_sh, V_sh, Wo)


# device time: 29462 ns/iter; 1.6655x vs baseline; 1.1932x over previous
import jax
import jax.numpy as jnp
from jax import lax
from jax.experimental import pallas as pl
from jax.experimental.pallas import tpu as pltpu

N_DEV = 16
LOG2_N = 4
B, SQ, SKV, HQ_SH, DH = 2, 128, 128, 4, 64
D_MODEL = 512


def kernel(x, Wq, K_ext, V_ext, Wo):
    my_i = lax.axis_index("i")
    K_sh = lax.dynamic_slice_in_dim(K_ext, my_i * HQ_SH, HQ_SH, axis=2)
    V_sh = lax.dynamic_slice_in_dim(V_ext, my_i * HQ_SH, HQ_SH, axis=2)
    K_sh = jnp.transpose(K_sh, (0, 2, 1, 3))
    V_sh = jnp.transpose(V_sh, (0, 2, 1, 3))

    def body(x_ref, wq_ref, k_ref, v_ref, wo_ref, out_ref,
             ctx_ref, send_ref, recv_ref, send_sems, recv_sems):
        me = lax.axis_index("i")

        partners = [me ^ (1 << r) for r in range(LOG2_N)]

        def make_rdma(r, c):
            return pltpu.make_async_remote_copy(
                src_ref=send_ref.at[r, c],
                dst_ref=recv_ref.at[r, c],
                send_sem=send_sems.at[r, c],
                recv_sem=recv_sems.at[r, c],
                device_id=(partners[r],),
                device_id_type=pl.DeviceIdType.MESH,
            )

        barrier = pltpu.get_barrier_semaphore()
        for r in range(LOG2_N):
            pl.semaphore_signal(barrier, inc=1, device_id=(partners[r],),
                                device_id_type=pl.DeviceIdType.MESH)
        pl.semaphore_wait(barrier, LOG2_N)

        wq = wq_ref[...].astype(jnp.bfloat16)
        wo = wo_ref[...].astype(jnp.bfloat16)
        ri = lax.broadcasted_iota(jnp.int32, (SQ, SKV), 0)
        ci = lax.broadcasted_iota(jnp.int32, (SQ, SKV), 1)
        qb = ri // 64
        kb = ci // 64
        mask = (qb == kb) | ((kb % 4) == (qb % 4))
        for b in range(B):
            xb = x_ref[b].astype(jnp.bfloat16)
            q_all = jnp.dot(xb, wq, preferred_element_type=jnp.float32)
            q_all = q_all.astype(jnp.bfloat16)
            for h in range(HQ_SH):
                q = q_all[:, h * DH:(h + 1) * DH]
                k = k_ref[b, h].astype(jnp.bfloat16)
                s = lax.dot_general(q, k, (((1,), (1,)), ((), ())),
                                    preferred_element_type=jnp.float32)
                s = s * 0.125
                s = jnp.where(mask, s, -1e9)
                m = jnp.max(s, axis=-1, keepdims=True)
                w = jnp.exp(s - m)
                w = w / jnp.sum(w, axis=-1, keepdims=True)
                v = v_ref[b, h].astype(jnp.bfloat16)
                ctx = jnp.dot(w.astype(jnp.bfloat16), v,
                              preferred_element_type=jnp.float32)
                ctx_ref[b, :, h * DH:(h + 1) * DH] = ctx.astype(jnp.bfloat16)
            out_ref[b] = jnp.dot(ctx_ref[b], wo,
                                 preferred_element_type=jnp.float32)
            send_ref[0, b] = out_ref[b].astype(jnp.bfloat16)
            make_rdma(0, b).start()

        for r in range(LOG2_N):
            for c in range(B):
                make_rdma(r, c).wait_recv()
                out_ref[c] = out_ref[c] + recv_ref[r, c].astype(jnp.float32)
                if r + 1 < LOG2_N:
                    send_ref[r + 1, c] = out_ref[c].astype(jnp.bfloat16)
                    make_rdma(r + 1, c).start()

        for r in range(LOG2_N):
            for c in range(B):
                make_rdma(r, c).wait_send()

    return pl.pallas_call(
        body,
        out_shape=jax.ShapeDtypeStruct((B, SQ, D_MODEL), jnp.float32),
        in_specs=[pl.BlockSpec(memory_space=pltpu.VMEM)] * 5,
        out_specs=pl.BlockSpec(memory_space=pltpu.VMEM),
        scratch_shapes=[
            pltpu.VMEM((B, SQ, HQ_SH * DH), jnp.bfloat16),
            pltpu.VMEM((LOG2_N, B, SQ, D_MODEL), jnp.bfloat16),
            pltpu.VMEM((LOG2_N, B, SQ, D_MODEL), jnp.bfloat16),
            pltpu.SemaphoreType.DMA((LOG2_N, B)),
            pltpu.SemaphoreType.DMA((LOG2_N, B)),
        ],
        compiler_params=pltpu.CompilerParams(collective_id=0),
    )(x, Wq, K_sh, V_sh, Wo)


# device time: 25820 ns/iter; 1.9005x vs baseline; 1.1411x over previous
import jax
import jax.numpy as jnp
from jax import lax
from jax.experimental import pallas as pl
from jax.experimental.pallas import tpu as pltpu

N_DEV = 16
LOG2_N = 4
B, SQ, SKV, HQ_SH, DH = 2, 128, 128, 4, 64
D_MODEL = 512


def kernel(x, Wq, K_ext, V_ext, Wo):
    my_i = lax.axis_index("i")
    K_sh = lax.dynamic_slice_in_dim(K_ext, my_i * HQ_SH, HQ_SH, axis=2)
    V_sh = lax.dynamic_slice_in_dim(V_ext, my_i * HQ_SH, HQ_SH, axis=2)
    K_sh = jnp.transpose(K_sh.astype(jnp.bfloat16), (0, 2, 1, 3))
    V_sh = jnp.transpose(V_sh.astype(jnp.bfloat16), (0, 2, 1, 3))
    x16 = x.astype(jnp.bfloat16)
    Wq16 = Wq.astype(jnp.bfloat16)
    Wo16 = Wo.astype(jnp.bfloat16)

    def body(x_ref, wq_ref, k_ref, v_ref, wo_ref, out_ref,
             ctx_ref, send_ref, recv_ref, send_sems, recv_sems):
        me = lax.axis_index("i")
        partners = [me ^ (1 << j) for j in range(LOG2_N)]

        def bit(p, c):
            return p if c == 0 else LOG2_N - 1 - p

        def make_rdma(p, c):
            return pltpu.make_async_remote_copy(
                src_ref=send_ref.at[p, c],
                dst_ref=recv_ref.at[p, c],
                send_sem=send_sems.at[p, c],
                recv_sem=recv_sems.at[p, c],
                device_id=(partners[bit(p, c)],),
                device_id_type=pl.DeviceIdType.MESH,
            )

        barrier = pltpu.get_barrier_semaphore()
        for j in range(LOG2_N):
            pl.semaphore_signal(barrier, inc=1, device_id=(partners[j],),
                                device_id_type=pl.DeviceIdType.MESH)
        pl.semaphore_wait(barrier, LOG2_N)

        wq = wq_ref[...]
        wo = wo_ref[...]
        ri = lax.broadcasted_iota(jnp.int32, (SQ, SKV), 0)
        ci = lax.broadcasted_iota(jnp.int32, (SQ, SKV), 1)
        qb = ri // 64
        kb = ci // 64
        mask = (qb == kb) | ((kb % 4) == (qb % 4))
        for b in range(B):
            xb = x_ref[b]
            q_all = jnp.dot(xb, wq, preferred_element_type=jnp.float32)
            q_all = q_all.astype(jnp.bfloat16)
            for h in range(HQ_SH):
                q = q_all[:, h * DH:(h + 1) * DH]
                k = k_ref[b, h]
                s = lax.dot_general(q, k, (((1,), (1,)), ((), ())),
                                    preferred_element_type=jnp.float32)
                s = s * 0.125
                s = jnp.where(mask, s, -1e9)
                m = jnp.max(s, axis=-1, keepdims=True)
                w = jnp.exp(s - m)
                w = w / jnp.sum(w, axis=-1, keepdims=True)
                v = v_ref[b, h]
                ctx = jnp.dot(w.astype(jnp.bfloat16), v,
                              preferred_element_type=jnp.float32)
                ctx_ref[b, :, h * DH:(h + 1) * DH] = ctx.astype(jnp.bfloat16)
            out_ref[b] = jnp.dot(ctx_ref[b], wo,
                                 preferred_element_type=jnp.float32)
            send_ref[0, b] = out_ref[b].astype(jnp.bfloat16)
            make_rdma(0, b).start()

        for p in range(LOG2_N):
            for c in range(B):
                make_rdma(p, c).wait_recv()
                out_ref[c] = out_ref[c] + recv_ref[p, c].astype(jnp.float32)
                if p + 1 < LOG2_N:
                    send_ref[p + 1, c] = out_ref[c].astype(jnp.bfloat16)
                    make_rdma(p + 1, c).start()

        for p in range(LOG2_N):
            for c in range(B):
                make_rdma(p, c).wait_send()

    return pl.pallas_call(
        body,
        out_shape=jax.ShapeDtypeStruct((B, SQ, D_MODEL), jnp.float32),
        in_specs=[pl.BlockSpec(memory_space=pltpu.VMEM)] * 5,
        out_specs=pl.BlockSpec(memory_space=pltpu.VMEM),
        scratch_shapes=[
            pltpu.VMEM((B, SQ, HQ_SH * DH), jnp.bfloat16),
            pltpu.VMEM((LOG2_N, B, SQ, D_MODEL), jnp.bfloat16),
            pltpu.VMEM((LOG2_N, B, SQ, D_MODEL), jnp.bfloat16),
            pltpu.SemaphoreType.DMA((LOG2_N, B)),
            pltpu.SemaphoreType.DMA((LOG2_N, B)),
        ],
        compiler_params=pltpu.CompilerParams(collective_id=0),
    )(x16, Wq16, K_sh, V_sh, Wo16)
